# baseline (speedup 1.0000x reference)
import functools

import jax
import jax.numpy as jnp
from jax import lax
from jax.experimental import pallas as pl
from jax.experimental.pallas import tpu as pltpu
from jax.experimental.pallas import tpu_sc as plsc

B = 100000
D = 128
C = 400
NSUB = 4
SUB = C // NSUB
NCHUNK = B // C
_info = plsc.get_sparse_core_info()
NC = _info.num_cores
NS = _info.num_subcores
NW = NC * NS
MAX_T = -(-NCHUNK // NW)

_mesh = plsc.VectorSubcoreMesh(core_axis_name="c", subcore_axis_name="s")


@functools.partial(
    pl.kernel,
    out_type=jax.ShapeDtypeStruct((B, D), jnp.float32),
    mesh=_mesh,
    scratch_types=[
        pltpu.VMEM((2, NSUB, SUB), jnp.int32),
        pltpu.VMEM((2, C, D), jnp.float32),
        pltpu.SemaphoreType.DMA,
        pltpu.SemaphoreType.DMA,
        pltpu.SemaphoreType.DMA,
    ],
)
def _emb_lookup(idx_hbm, table_hbm, out_hbm, idx_v, rows_v, gsem, ssem0, ssem1):
    wid = lax.axis_index("s") * NC + lax.axis_index("c")
    ssems = (ssem0, ssem1)

    def wait_scatter(b):
        pltpu.make_async_copy(rows_v.at[b], out_hbm.at[pl.ds(0, C)], ssems[b]).wait()

    for t in range(MAX_T):
        b = t % 2
        chunk = wid + t * NW

        @pl.when(chunk < NCHUNK)
        def _():
            if t >= 2:
                wait_scatter(b)
            pltpu.sync_copy(idx_hbm.at[chunk], idx_v.at[b])
            copies = [
                pltpu.async_copy(
                    table_hbm.at[idx_v.at[b, j]],
                    rows_v.at[b, pl.ds(j * SUB, SUB)],
                    gsem,
                )
                for j in range(NSUB)
            ]
            for cp in copies:
                cp.wait()
            pltpu.async_copy(rows_v.at[b], out_hbm.at[pl.ds(chunk * C, C)], ssems[b])

    wait_scatter(0)
    wait_scatter(1)


def kernel(node_type_indices, table):
    idx = node_type_indices.astype(jnp.int32).reshape(NCHUNK, NSUB, SUB)
    return _emb_lookup(idx, table)


# speedup vs baseline: 7.3297x; 7.3297x over previous
import functools

import jax
import jax.numpy as jnp
from jax import lax
from jax.experimental import pallas as pl
from jax.experimental.pallas import tpu as pltpu
from jax.experimental.pallas import tpu_sc as plsc

B = 100000
D = 128
C = 400
NSUB = 4
SUB = C // NSUB
NCHUNK = B // C
_info = plsc.get_sparse_core_info()
NC = _info.num_cores
NS = _info.num_subcores
NW = NC * NS
MAX_T = -(-NCHUNK // NW)

_mesh = plsc.VectorSubcoreMesh(core_axis_name="c", subcore_axis_name="s")


@functools.partial(
    pl.kernel,
    out_type=jax.ShapeDtypeStruct((B, D), jnp.float32),
    mesh=_mesh,
    scratch_types=[
        pltpu.VMEM((2, NSUB, SUB), jnp.int32),
        pltpu.VMEM((2, C, D), jnp.float32),
        pltpu.VMEM_SHARED((16, D), jnp.float32),
        pltpu.SemaphoreType.DMA,
        pltpu.SemaphoreType.DMA,
        pltpu.SemaphoreType.DMA,
    ],
)
def _emb_lookup(idx_hbm, table_hbm, out_hbm, idx_v, rows_v, table_sh,
                gsem, ssem0, ssem1):
    sid = lax.axis_index("s")
    wid = sid * NC + lax.axis_index("c")
    ssems = (ssem0, ssem1)

    @pl.when(sid == 0)
    def _():
        pltpu.sync_copy(table_hbm, rows_v.at[0, pl.ds(0, 16)])
        pltpu.sync_copy(rows_v.at[0, pl.ds(0, 16)], table_sh)

    plsc.subcore_barrier()

    def wait_scatter(b):
        pltpu.make_async_copy(rows_v.at[b], out_hbm.at[pl.ds(0, C)], ssems[b]).wait()

    for t in range(MAX_T):
        b = t % 2
        chunk = wid + t * NW

        @pl.when(chunk < NCHUNK)
        def _():
            if t >= 2:
                wait_scatter(b)
            pltpu.sync_copy(idx_hbm.at[chunk], idx_v.at[b])
            copies = [
                pltpu.async_copy(
                    table_sh.at[idx_v.at[b, j]],
                    rows_v.at[b, pl.ds(j * SUB, SUB)],
                    gsem,
                )
                for j in range(NSUB)
            ]
            for cp in copies:
                cp.wait()
            pltpu.async_copy(rows_v.at[b], out_hbm.at[pl.ds(chunk * C, C)], ssems[b])

    wait_scatter(0)
    wait_scatter(1)


def kernel(node_type_indices, table):
    idx = node_type_indices.astype(jnp.int32).reshape(NCHUNK, NSUB, SUB)
    return _emb_lookup(idx, table)


# speedup vs baseline: 8.0074x; 1.0925x over previous
import functools

import jax
import jax.numpy as jnp
from jax import lax
from jax.experimental import pallas as pl
from jax.experimental.pallas import tpu as pltpu
from jax.experimental.pallas import tpu_sc as plsc

B = 100000
D = 128
C = 400
NSUB = 4
SUB = C // NSUB
NCHUNK = B // C
_info = plsc.get_sparse_core_info()
NC = _info.num_cores
NS = _info.num_subcores
NW = NC * NS
MAX_T = -(-NCHUNK // NW)
NFULL = NCHUNK - NW * (MAX_T - 1)

_mesh = plsc.VectorSubcoreMesh(core_axis_name="c", subcore_axis_name="s")


@functools.partial(
    pl.kernel,
    out_type=jax.ShapeDtypeStruct((B, D), jnp.float32),
    mesh=_mesh,
    scratch_types=[
        pltpu.VMEM((MAX_T, NSUB, SUB), jnp.int32),
        pltpu.VMEM((2, C, D), jnp.float32),
        pltpu.VMEM_SHARED((16, D), jnp.float32),
        pltpu.SemaphoreType.DMA,
        pltpu.SemaphoreType.DMA,
        pltpu.SemaphoreType.DMA,
    ],
)
def _emb_lookup(idx_hbm, table_hbm, out_hbm, idx_v, rows_v, table_sh,
                gsem, ssem0, ssem1):
    sid = lax.axis_index("s")
    wid = sid * NC + lax.axis_index("c")
    start = jnp.where(wid < NFULL, wid * MAX_T,
                      NFULL * MAX_T + (wid - NFULL) * (MAX_T - 1))
    ssems = (ssem0, ssem1)

    pltpu.sync_copy(idx_hbm.at[pl.ds(start, MAX_T)], idx_v)

    @pl.when(sid == 0)
    def _():
        pltpu.sync_copy(table_hbm, rows_v.at[0, pl.ds(0, 16)])
        pltpu.sync_copy(rows_v.at[0, pl.ds(0, 16)], table_sh)

    plsc.subcore_barrier()

    def wait_scatter(b):
        pltpu.make_async_copy(rows_v.at[b], out_hbm.at[pl.ds(0, C)], ssems[b]).wait()

    def do_chunk(t):
        b = t % 2
        if t >= 2:
            wait_scatter(b)
        copies = [
            pltpu.async_copy(
                table_sh.at[idx_v.at[t, j]],
                rows_v.at[b, pl.ds(j * SUB, SUB)],
                gsem,
            )
            for j in range(NSUB)
        ]
        for cp in copies:
            cp.wait()
        pltpu.async_copy(rows_v.at[b], out_hbm.at[pl.ds((start + t) * C, C)],
                         ssems[b])

    for t in range(MAX_T - 1):
        do_chunk(t)

    @pl.when(wid < NFULL)
    def _():
        do_chunk(MAX_T - 1)

    wait_scatter(0)
    wait_scatter(1)


def kernel(node_type_indices, table):
    idx = node_type_indices.astype(jnp.int32).reshape(NCHUNK, C)
    idx = jnp.concatenate([idx, jnp.zeros((1, C), jnp.int32)], axis=0)
    idx = idx.reshape(NCHUNK + 1, NSUB, SUB)
    return _emb_lookup(idx, table)
